# baseline (speedup 1.0000x reference)
import jax
import jax.numpy as jnp
from jax import lax
from jax.experimental import pallas as pl
from jax.experimental.pallas import tpu as pltpu


def _make_kernel(rows_per_chunk, t_tgt):
    def _kernel(src_ref, tgt_ref, w_enc_ref, b_enc_ref, w_out_ref,
                b_out_ref, o_ref, ctx_ref, h_ref, w_bf_ref):
        j = pl.program_id(0)
        m = pl.program_id(1)
        mc = rows_per_chunk * t_tgt

        @pl.when(j == 0)
        def _():
            sl = pl.ds(m * mc, mc)
            ctx_ref[sl, :] = jnp.tanh(
                jnp.dot(src_ref[sl, :], w_enc_ref[...],
                        preferred_element_type=jnp.float32)
                + b_enc_ref[...]).astype(jnp.bfloat16)

            for i in range(rows_per_chunk):
                rl = pl.ds(m * mc + i * t_tgt, t_tgt)
                ctx = ctx_ref[rl, :]
                e = tgt_ref[rl, :]
                scores = lax.dot_general(
                    e.astype(jnp.bfloat16), ctx, (((1,), (1,)), ((), ())),
                    preferred_element_type=jnp.float32)
                mx = jnp.max(scores, axis=-1, keepdims=True)
                p = jnp.exp(scores - mx)
                p = p / jnp.sum(p, axis=-1, keepdims=True)
                attn = jnp.dot(p.astype(jnp.bfloat16), ctx,
                               preferred_element_type=jnp.float32)
                h_ref[rl, :] = (e + attn).astype(jnp.bfloat16)

        @pl.when(m == 0)
        def _():
            w_bf_ref[...] = w_out_ref[...].astype(jnp.bfloat16)

        o_ref[...] = (
            jnp.dot(h_ref[pl.ds(m * mc, mc), :], w_bf_ref[...],
                    preferred_element_type=jnp.float32)
            + b_out_ref[...])

    return _kernel


def kernel(enc_emb, dec_emb, w_enc, b_enc, w_out, b_out, src, tgt):
    src_emb = enc_emb[src]
    tgt_emb = dec_emb[tgt]

    B, T_src, E = src_emb.shape
    _, T_tgt, H = tgt_emb.shape
    V = w_out.shape[1]

    src_flat = src_emb.reshape(B * T_src, E)
    tgt_flat = tgt_emb.reshape(B * T_tgt, H)

    tile_v = min(2048, V)
    n_vt = V // tile_v
    rows_per_chunk = 8 if B % 8 == 0 else B
    n_mc = B // rows_per_chunk
    m_tile = rows_per_chunk * T_tgt

    logits = pl.pallas_call(
        _make_kernel(rows_per_chunk, T_tgt),
        out_shape=jax.ShapeDtypeStruct((B * T_tgt, V), jnp.float32),
        grid=(n_vt, n_mc),
        in_specs=[
            pl.BlockSpec((B * T_src, E), lambda j, m: (0, 0)),
            pl.BlockSpec((B * T_tgt, H), lambda j, m: (0, 0)),
            pl.BlockSpec((E, H), lambda j, m: (0, 0)),
            pl.BlockSpec((1, H), lambda j, m: (0, 0)),
            pl.BlockSpec((H, tile_v), lambda j, m: (0, j)),
            pl.BlockSpec((1, tile_v), lambda j, m: (0, j)),
        ],
        out_specs=pl.BlockSpec((m_tile, tile_v), lambda j, m: (m, j)),
        scratch_shapes=[
            pltpu.VMEM((B * T_src, H), jnp.bfloat16),
            pltpu.VMEM((B * T_tgt, H), jnp.bfloat16),
            pltpu.VMEM((H, tile_v), jnp.bfloat16),
        ],
        compiler_params=pltpu.CompilerParams(
            dimension_semantics=("arbitrary", "arbitrary")),
    )(src_flat, tgt_flat, w_enc, b_enc, w_out, b_out)

    return logits.reshape(B, T_tgt, V)


# speedup vs baseline: 1.1391x; 1.1391x over previous
import jax
import jax.numpy as jnp
from jax import lax
from jax.experimental import pallas as pl
from jax.experimental.pallas import tpu as pltpu


def _make_kernel(n_rows, t_tgt):
    def _kernel(src_ref, tgt_ref, w_enc_ref, b_enc_ref, w_out_ref,
                b_out_ref, o_ref, ctx_ref, h_ref):
        j = pl.program_id(0)

        @pl.when(j == 0)
        def _():
            ctx_ref[...] = jnp.tanh(
                jnp.dot(src_ref[...], w_enc_ref[...],
                        preferred_element_type=jnp.float32)
                + b_enc_ref[...]).astype(jnp.bfloat16)

            for i in range(n_rows):
                sl = pl.ds(i * t_tgt, t_tgt)
                ctx = ctx_ref[sl, :]
                e = tgt_ref[sl, :]
                scores = lax.dot_general(
                    e.astype(jnp.bfloat16), ctx, (((1,), (1,)), ((), ())),
                    preferred_element_type=jnp.float32)
                m = jnp.max(scores, axis=-1, keepdims=True)
                p = jnp.exp(scores - m)
                p = p / jnp.sum(p, axis=-1, keepdims=True)
                attn = jnp.dot(p.astype(jnp.bfloat16), ctx,
                               preferred_element_type=jnp.float32)
                h_ref[sl, :] = (e + attn).astype(jnp.bfloat16)

        w = w_out_ref[...].astype(jnp.bfloat16)
        o_ref[...] = (
            jnp.dot(h_ref[...], w, preferred_element_type=jnp.float32)
            + b_out_ref[...])

    return _kernel


def kernel(enc_emb, dec_emb, w_enc, b_enc, w_out, b_out, src, tgt):
    src_emb = enc_emb.at[src.reshape(-1)].get(mode="promise_in_bounds")
    tgt_emb = dec_emb.at[tgt.reshape(-1)].get(mode="promise_in_bounds")

    B, T_src = src.shape
    _, T_tgt = tgt.shape
    E = enc_emb.shape[1]
    H = dec_emb.shape[1]
    V = w_out.shape[1]

    tile_v = min(2048, V)
    n_vt = V // tile_v

    logits = pl.pallas_call(
        _make_kernel(B, T_tgt),
        out_shape=jax.ShapeDtypeStruct((B * T_tgt, V), jnp.float32),
        grid=(n_vt,),
        in_specs=[
            pl.BlockSpec((B * T_src, E), lambda j: (0, 0)),
            pl.BlockSpec((B * T_tgt, H), lambda j: (0, 0)),
            pl.BlockSpec((E, H), lambda j: (0, 0)),
            pl.BlockSpec((1, H), lambda j: (0, 0)),
            pl.BlockSpec((H, tile_v), lambda j: (0, j)),
            pl.BlockSpec((1, tile_v), lambda j: (0, j)),
        ],
        out_specs=pl.BlockSpec((B * T_tgt, tile_v), lambda j: (0, j)),
        scratch_shapes=[
            pltpu.VMEM((B * T_src, H), jnp.bfloat16),
            pltpu.VMEM((B * T_tgt, H), jnp.bfloat16),
        ],
        compiler_params=pltpu.CompilerParams(
            dimension_semantics=("arbitrary",)),
    )(src_emb, tgt_emb, w_enc, b_enc, w_out, b_out)

    return logits.reshape(B, T_tgt, V)
